# baseline (speedup 1.0000x reference)
import functools

import jax
import jax.numpy as jnp
from jax import lax
from jax.experimental import pallas as pl
from jax.experimental.pallas import tpu as pltpu
from jax.experimental.pallas import tpu_sc as plsc

L = 16
NC = 2
NS = 16
NW = NC * NS
K = 8
LWIN = 16

_CORNERS = [(dx, dy, dz) for dx in (0, 1) for dy in (0, 1) for dz in (0, 1)]


def _body(side, npt, c, data_hbm, lut_hbm, xs_hbm, ys_hbm, zs_hbm, out_hbm,
          xs_v, ys_v, zs_v, flat_v, node_v, w_v,
          rows0_v, rows1_v, out0_v, out1_v, lsem, rsem, osem):
    PW = npt // NW
    G = PW // L
    KL = K * L
    CG = c // L
    scale = side * 0.5

    rows_b = (rows0_v, rows1_v)
    out_b = (out0_v, out1_v)

    wid = lax.axis_index("s") * NC + lax.axis_index("c")
    base = wid * PW

    pltpu.sync_copy(xs_hbm.at[pl.ds(base, PW)], xs_v)
    pltpu.sync_copy(ys_hbm.at[pl.ds(base, PW)], ys_v)
    pltpu.sync_copy(zs_hbm.at[pl.ds(base, PW)], zs_v)

    zv = jnp.zeros((2 * L,), jnp.bfloat16)

    def zinit(r, carry):
        for q in range(CG // 2):
            rows0_v[r, pl.ds(q * 2 * L, 2 * L)] = zv
            rows1_v[r, pl.ds(q * 2 * L, 2 * L)] = zv
        return carry

    lax.fori_loop(0, KL, zinit, 0)

    def lut_wait():
        pltpu.make_async_copy(lut_hbm.at[flat_v.at[0]], node_v.at[0], lsem).wait()

    def fsplit(v):
        vf = (v + 1.0) * scale - 0.5
        vi = (vf + 1.0).astype(jnp.int32) - 1
        vi = jnp.where(vi.astype(jnp.float32) > vf, vi - 1, vi)
        fr = vf - vi.astype(jnp.float32)
        return vi, fr

    def phase1(g, carry):
        x = xs_v[pl.ds(g * L, L)]
        y = ys_v[pl.ds(g * L, L)]
        z = zs_v[pl.ds(g * L, L)]
        xi, fx = fsplit(x)
        yi, fy = fsplit(y)
        zi, fz = fsplit(z)
        for k, (dx, dy, dz) in enumerate(_CORNERS):
            cx = xi + dx
            cy = yi + dy
            cz = zi + dz
            inb = ((cx >= 0) & (cx < side) & (cy >= 0) & (cy < side)
                   & (cz >= 0) & (cz < side))
            ccx = jnp.clip(cx, 0, side - 1)
            ccy = jnp.clip(cy, 0, side - 1)
            ccz = jnp.clip(cz, 0, side - 1)
            flat = (ccx * side + ccy) * side + ccz
            w = jnp.abs(((1 - dx) - fx) * ((1 - dy) - fy) * ((1 - dz) - fz))
            w = jnp.where(inb, w, 0.0)
            flat_v[g, pl.ds(k * L, L)] = flat
            w_v[g, pl.ds(k * L, L)] = w
        pltpu.async_copy(lut_hbm.at[flat_v.at[g]], node_v.at[g], lsem)

        @pl.when(g >= LWIN)
        def _():
            lut_wait()
        return carry

    lax.fori_loop(0, G, phase1, 0)

    def drain_luts(_, carry):
        lut_wait()
        return carry

    lax.fori_loop(0, min(LWIN, G), drain_luts, 0)

    def corner_fire_idx(h, k):
        nd = node_v[h, pl.ds(k * L, L)]
        wv = w_v[h, pl.ds(k * L, L)]
        return jnp.where((nd > -1) & (wv > 0.0), nd, -1)

    def stage_a(h, bn):
        rows = rows_b[bn]
        for k in range(K):
            widx = corner_fire_idx(h, k)
            wv = w_v[h, pl.ds(k * L, L)]
            w_v[h, pl.ds(k * L, L)] = jnp.where(widx > -1, wv, 0.0)
            for j in range(L):
                idx = widx[j]

                @pl.when(idx > -1)
                def _():
                    pltpu.async_copy(data_hbm.at[pl.ds(idx, 1)],
                                     rows.at[pl.ds(k * L + j, 1)], rsem)

    def rows_wait(b, g):
        for k in range(K):
            widx = corner_fire_idx(g, k)
            for j in range(L):
                idx = widx[j]

                @pl.when(idx > -1)
                def _():
                    pltpu.make_async_copy(data_hbm.at[pl.ds(0, 1)],
                                          rows_b[b].at[pl.ds(0, 1)],
                                          rsem).wait()

    def stage_c(g, b):
        rows = rows_b[b]
        outv = out_b[b]

        def pbody(p, carry):
            ws = [plsc.load_gather(
                      w_v, [jnp.full((L,), g, jnp.int32),
                            jnp.full((L,), k * L + p, jnp.int32)])
                  for k in range(K)]
            nrm = ws[0]
            for k in range(1, K):
                nrm = nrm + ws[k]
            inv = 1.0 / (nrm + 1e-12)
            for q in range(CG // 2):
                ab = rows[p, pl.ds(q * 2 * L, 2 * L)]
                a, bb = plsc.unpack(ab, format=plsc.PackFormat.INTERLEAVED)
                acc_a = ws[0] * a
                acc_b = ws[0] * bb
                for k in range(1, K):
                    ab = rows[k * L + p, pl.ds(q * 2 * L, 2 * L)]
                    a, bb = plsc.unpack(ab, format=plsc.PackFormat.INTERLEAVED)
                    acc_a = acc_a + ws[k] * a
                    acc_b = acc_b + ws[k] * bb
                outv[p, pl.ds(q * 2 * L, L)] = acc_a * inv
                outv[p, pl.ds(q * 2 * L + L, L)] = acc_b * inv
            return carry

        lax.fori_loop(0, L, pbody, 0)
        pltpu.async_copy(outv, out_hbm.at[pl.ds(base + g * L, L)], osem)

    def out_wait(b):
        pltpu.make_async_copy(out_b[b], out_hbm.at[pl.ds(base, L)], osem).wait()

    stage_a(0, 0)

    def main(i, carry):
        for off in range(2):
            g = i * 2 + off
            b = off
            bn = 1 - off

            @pl.when(g + 1 < G)
            def _():
                stage_a(g + 1, bn)

            rows_wait(b, g)

            @pl.when(g >= 2)
            def _():
                out_wait(b)

            stage_c(g, b)
        return carry

    lax.fori_loop(0, G // 2, main, 0)
    out_wait(0)
    out_wait(1)


@functools.partial(jax.jit, static_argnums=(2, 3, 4))
def _interp(data, lut, npt, c, side, xs, ys, zs):
    PW = npt // NW
    G = PW // L
    KL = K * L
    mesh = plsc.VectorSubcoreMesh(core_axis_name="c", subcore_axis_name="s")
    kern = pl.kernel(
        functools.partial(_body, side, npt, c),
        out_type=jax.ShapeDtypeStruct((npt, c), jnp.float32),
        mesh=mesh,
        scratch_types=[
            pltpu.VMEM((PW,), jnp.float32),
            pltpu.VMEM((PW,), jnp.float32),
            pltpu.VMEM((PW,), jnp.float32),
            pltpu.VMEM((G, KL), jnp.int32),
            pltpu.VMEM((G, KL), jnp.int32),
            pltpu.VMEM((G, KL), jnp.float32),
            pltpu.VMEM((KL, c), jnp.bfloat16),
            pltpu.VMEM((KL, c), jnp.bfloat16),
            pltpu.VMEM((L, c), jnp.float32),
            pltpu.VMEM((L, c), jnp.float32),
            pltpu.SemaphoreType.DMA,
            pltpu.SemaphoreType.DMA,
            pltpu.SemaphoreType.DMA,
        ],
        compiler_params=pltpu.CompilerParams(
            needs_layout_passes=False, use_tc_tiling_on_sc=False),
    )
    return kern(data, lut, xs, ys, zs)


def kernel(data, octree_lookup, depth, pts):
    del depth
    npt = pts.shape[0]
    c = data.shape[1]
    nvox = octree_lookup.shape[0]
    side = round(nvox ** (1.0 / 3.0))
    assert side ** 3 == nvox and npt % (NW * L) == 0 and c % L == 0
    xs = pts[:, 0]
    ys = pts[:, 1]
    zs = pts[:, 2]
    data_bf = (data.reshape(nnum_rows(data), c // 32, 2, 16)
               .swapaxes(2, 3).reshape(data.shape[0], c).astype(jnp.bfloat16))
    return _interp(data_bf, octree_lookup, npt, c, side, xs, ys, zs)


def nnum_rows(data):
    return data.shape[0]


# speedup vs baseline: 2.6201x; 2.6201x over previous
import functools

import jax
import jax.numpy as jnp
from jax import lax
from jax.experimental import pallas as pl
from jax.experimental.pallas import tpu as pltpu
from jax.experimental.pallas import tpu_sc as plsc

L = 16
NC = 2
NS = 16
NW = NC * NS
K = 8
LWIN = 16

_CORNERS = [(dx, dy, dz) for dx in (0, 1) for dy in (0, 1) for dz in (0, 1)]


def _body(side, npt, c, data_hbm, lut_hbm, xs_hbm, ys_hbm, zs_hbm, out_hbm,
          xs_v, ys_v, zs_v, flat_v, node_v, w_v, ridx0_v, ridx1_v,
          rows0_v, rows1_v, out0_v, out1_v, lsem, rsem, osem):
    PW = npt // NW
    G = PW // L
    KL = K * L
    CG = c // L
    scale = side * 0.5

    ridx_b = (ridx0_v, ridx1_v)
    rows_b = (rows0_v, rows1_v)
    out_b = (out0_v, out1_v)

    wid = lax.axis_index("s") * NC + lax.axis_index("c")
    base = wid * PW

    pltpu.sync_copy(xs_hbm.at[pl.ds(base, PW)], xs_v)
    pltpu.sync_copy(ys_hbm.at[pl.ds(base, PW)], ys_v)
    pltpu.sync_copy(zs_hbm.at[pl.ds(base, PW)], zs_v)

    zv = jnp.zeros((L,), jnp.float32)

    def zinit(r, carry):
        for cg in range(CG):
            rows0_v[r, pl.ds(cg * L, L)] = zv
            rows1_v[r, pl.ds(cg * L, L)] = zv
        return carry

    lax.fori_loop(0, KL, zinit, 0)

    def lut_wait():
        pltpu.make_async_copy(lut_hbm.at[flat_v.at[0]], node_v.at[0], lsem).wait()

    def fsplit(v):
        vf = (v + 1.0) * scale - 0.5
        vi = (vf + 1.0).astype(jnp.int32) - 1
        vi = jnp.where(vi.astype(jnp.float32) > vf, vi - 1, vi)
        fr = vf - vi.astype(jnp.float32)
        return vi, fr

    def phase1(g, carry):
        x = xs_v[pl.ds(g * L, L)]
        y = ys_v[pl.ds(g * L, L)]
        z = zs_v[pl.ds(g * L, L)]
        xi, fx = fsplit(x)
        yi, fy = fsplit(y)
        zi, fz = fsplit(z)
        for k, (dx, dy, dz) in enumerate(_CORNERS):
            cx = xi + dx
            cy = yi + dy
            cz = zi + dz
            inb = ((cx >= 0) & (cx < side) & (cy >= 0) & (cy < side)
                   & (cz >= 0) & (cz < side))
            ccx = jnp.clip(cx, 0, side - 1)
            ccy = jnp.clip(cy, 0, side - 1)
            ccz = jnp.clip(cz, 0, side - 1)
            flat = (ccx * side + ccy) * side + ccz
            w = jnp.abs(((1 - dx) - fx) * ((1 - dy) - fy) * ((1 - dz) - fz))
            w = jnp.where(inb, w, 0.0)
            flat_v[g, pl.ds(k * L, L)] = flat
            w_v[g, pl.ds(k * L, L)] = w
        pltpu.async_copy(lut_hbm.at[flat_v.at[g]], node_v.at[g], lsem)

        @pl.when(g >= LWIN)
        def _():
            lut_wait()
        return carry

    lax.fori_loop(0, G, phase1, 0)

    def drain_luts(_, carry):
        lut_wait()
        return carry

    lax.fori_loop(0, min(LWIN, G), drain_luts, 0)

    def stage_a(h, bn):
        for k in range(K):
            nd = node_v[h, pl.ds(k * L, L)]
            valid = nd > -1
            dummy = jnp.bitwise_and(flat_v[h, pl.ds(k * L, L)],
                                    jnp.int32(npt - 1))
            ridx_b[bn][pl.ds(k * L, L)] = jnp.where(valid, nd, dummy)
            wv = w_v[h, pl.ds(k * L, L)]
            w_v[h, pl.ds(k * L, L)] = jnp.where(valid, wv, 0.0)
        pltpu.async_copy(data_hbm.at[ridx_b[bn]], rows_b[bn], rsem)

    def rows_wait(b, g):
        del g
        pltpu.make_async_copy(data_hbm.at[ridx_b[b]], rows_b[b], rsem).wait()

    def stage_c(g, b):
        rows = rows_b[b]
        outv = out_b[b]

        def pbody(p, carry):
            ws = [plsc.load_gather(
                      w_v, [jnp.full((L,), g, jnp.int32),
                            jnp.full((L,), k * L + p, jnp.int32)])
                  for k in range(K)]
            nrm = ws[0]
            for k in range(1, K):
                nrm = nrm + ws[k]
            inv = 1.0 / (nrm + 1e-12)
            for cg in range(CG):
                acc = ws[0] * rows[p, pl.ds(cg * L, L)]
                for k in range(1, K):
                    acc = acc + ws[k] * rows[k * L + p, pl.ds(cg * L, L)]
                outv[p, pl.ds(cg * L, L)] = acc * inv
            return carry

        lax.fori_loop(0, L, pbody, 0)
        pltpu.async_copy(outv, out_hbm.at[pl.ds(base + g * L, L)], osem)

    def out_wait(b):
        pltpu.make_async_copy(out_b[b], out_hbm.at[pl.ds(base, L)], osem).wait()

    stage_a(0, 0)

    def main(i, carry):
        for off in range(2):
            g = i * 2 + off
            b = off
            bn = 1 - off

            @pl.when(g + 1 < G)
            def _():
                stage_a(g + 1, bn)

            rows_wait(b, g)

            @pl.when(g >= 2)
            def _():
                out_wait(b)

            stage_c(g, b)
        return carry

    lax.fori_loop(0, G // 2, main, 0)
    out_wait(0)
    out_wait(1)


@functools.partial(jax.jit, static_argnums=(2, 3, 4))
def _interp(data, lut, npt, c, side, xs, ys, zs):
    PW = npt // NW
    G = PW // L
    KL = K * L
    mesh = plsc.VectorSubcoreMesh(core_axis_name="c", subcore_axis_name="s")
    kern = pl.kernel(
        functools.partial(_body, side, npt, c),
        out_type=jax.ShapeDtypeStruct((npt, c), jnp.float32),
        mesh=mesh,
        scratch_types=[
            pltpu.VMEM((PW,), jnp.float32),
            pltpu.VMEM((PW,), jnp.float32),
            pltpu.VMEM((PW,), jnp.float32),
            pltpu.VMEM((G, KL), jnp.int32),
            pltpu.VMEM((G, KL), jnp.int32),
            pltpu.VMEM((G, KL), jnp.float32),
            pltpu.VMEM((KL,), jnp.int32),
            pltpu.VMEM((KL,), jnp.int32),
            pltpu.VMEM((KL, c), jnp.float32),
            pltpu.VMEM((KL, c), jnp.float32),
            pltpu.VMEM((L, c), jnp.float32),
            pltpu.VMEM((L, c), jnp.float32),
            pltpu.SemaphoreType.DMA,
            pltpu.SemaphoreType.DMA,
            pltpu.SemaphoreType.DMA,
        ],
        compiler_params=pltpu.CompilerParams(
            needs_layout_passes=False, use_tc_tiling_on_sc=False),
    )
    return kern(data, lut, xs, ys, zs)


def kernel(data, octree_lookup, depth, pts):
    del depth
    npt = pts.shape[0]
    c = data.shape[1]
    nvox = octree_lookup.shape[0]
    side = round(nvox ** (1.0 / 3.0))
    assert side ** 3 == nvox and npt % (NW * L) == 0 and c % L == 0
    xs = pts[:, 0]
    ys = pts[:, 1]
    zs = pts[:, 2]
    return _interp(data, octree_lookup, npt, c, side, xs, ys, zs)


# speedup vs baseline: 2.7613x; 1.0539x over previous
import functools

import jax
import jax.numpy as jnp
from jax import lax
from jax.experimental import pallas as pl
from jax.experimental.pallas import tpu as pltpu
from jax.experimental.pallas import tpu_sc as plsc

L = 16
NC = 2
NS = 16
NW = NC * NS
K = 8
LWIN = 16

_CORNERS = [(dx, dy, dz) for dx in (0, 1) for dy in (0, 1) for dz in (0, 1)]


def _body(side, npt, c, data_hbm, lut_hbm, xs_hbm, ys_hbm, zs_hbm, out_hbm,
          xs_v, ys_v, zs_v, flat_v, node_v, w_v, ridx0_v, ridx1_v,
          rows0_v, rows1_v, out0_v, out1_v, lsem, rsem, osem):
    PW = npt // NW
    G = PW // L
    KL = K * L
    CG = c // L
    scale = side * 0.5

    ridx_b = (ridx0_v, ridx1_v)
    rows_b = (rows0_v, rows1_v)
    out_b = (out0_v, out1_v)

    wid = lax.axis_index("s") * NC + lax.axis_index("c")
    base = wid * PW

    pltpu.sync_copy(xs_hbm.at[pl.ds(base, PW)], xs_v)
    pltpu.sync_copy(ys_hbm.at[pl.ds(base, PW)], ys_v)
    pltpu.sync_copy(zs_hbm.at[pl.ds(base, PW)], zs_v)

    zv = jnp.zeros((L,), jnp.float32)

    def zinit(r, carry):
        for cg in range(CG):
            rows0_v[r, pl.ds(cg * L, L)] = zv
            rows1_v[r, pl.ds(cg * L, L)] = zv
        return carry

    lax.fori_loop(0, KL, zinit, 0)

    def lut_wait():
        pltpu.make_async_copy(lut_hbm.at[flat_v.at[0]], node_v.at[0], lsem).wait()

    def fsplit(v):
        vf = (v + 1.0) * scale - 0.5
        vi = (vf + 1.0).astype(jnp.int32) - 1
        vi = jnp.where(vi.astype(jnp.float32) > vf, vi - 1, vi)
        fr = vf - vi.astype(jnp.float32)
        return vi, fr

    def phase1(g):
        x = xs_v[pl.ds(g * L, L)]
        y = ys_v[pl.ds(g * L, L)]
        z = zs_v[pl.ds(g * L, L)]
        xi, fx = fsplit(x)
        yi, fy = fsplit(y)
        zi, fz = fsplit(z)
        for k, (dx, dy, dz) in enumerate(_CORNERS):
            cx = xi + dx
            cy = yi + dy
            cz = zi + dz
            inb = ((cx >= 0) & (cx < side) & (cy >= 0) & (cy < side)
                   & (cz >= 0) & (cz < side))
            ccx = jnp.clip(cx, 0, side - 1)
            ccy = jnp.clip(cy, 0, side - 1)
            ccz = jnp.clip(cz, 0, side - 1)
            flat = (ccx * side + ccy) * side + ccz
            w = jnp.abs(((1 - dx) - fx) * ((1 - dy) - fy) * ((1 - dz) - fz))
            w = jnp.where(inb, w, 0.0)
            flat_v[g, pl.ds(k * L, L)] = flat
            w_v[g, pl.ds(k * L, L)] = w
        pltpu.async_copy(lut_hbm.at[flat_v.at[g]], node_v.at[g], lsem)

    def stage_a(h, bn):
        for k in range(K):
            nd = node_v[h, pl.ds(k * L, L)]
            valid = nd > -1
            dummy = jnp.bitwise_and(flat_v[h, pl.ds(k * L, L)],
                                    jnp.int32(npt - 1))
            ridx_b[bn][pl.ds(k * L, L)] = jnp.where(valid, nd, dummy)
            wv = w_v[h, pl.ds(k * L, L)]
            w_v[h, pl.ds(k * L, L)] = jnp.where(valid, wv, 0.0)
        pltpu.async_copy(data_hbm.at[ridx_b[bn]], rows_b[bn], rsem)

    def rows_wait(b, g):
        del g
        pltpu.make_async_copy(data_hbm.at[ridx_b[b]], rows_b[b], rsem).wait()

    def stage_c(g, b):
        rows = rows_b[b]
        outv = out_b[b]

        def pbody(p, carry):
            ws = [plsc.load_gather(
                      w_v, [jnp.full((L,), g, jnp.int32),
                            jnp.full((L,), k * L + p, jnp.int32)])
                  for k in range(K)]
            nrm = ws[0]
            for k in range(1, K):
                nrm = nrm + ws[k]
            inv = 1.0 / (nrm + 1e-12)
            for cg in range(CG):
                acc = ws[0] * rows[p, pl.ds(cg * L, L)]
                for k in range(1, K):
                    acc = acc + ws[k] * rows[k * L + p, pl.ds(cg * L, L)]
                outv[p, pl.ds(cg * L, L)] = acc * inv
            return carry

        lax.fori_loop(0, L, pbody, 0)
        pltpu.async_copy(outv, out_hbm.at[pl.ds(base + g * L, L)], osem)

    def out_wait(b):
        pltpu.make_async_copy(out_b[b], out_hbm.at[pl.ds(base, L)], osem).wait()

    SB = 16
    NSB = G // SB

    def fire_sb(s):
        def fb(g, carry):
            phase1(g)
            return carry
        lax.fori_loop(s * SB, (s + 1) * SB, fb, 0)

    def drain_sb():
        def db(_, carry):
            lut_wait()
            return carry
        lax.fori_loop(0, SB, db, 0)

    fire_sb(0)
    drain_sb()
    stage_a(0, 0)

    def main(i, carry):
        for off in range(2):
            g = i * 2 + off
            b = off
            bn = 1 - off

            @pl.when(g % SB == 0)
            def _():
                s = g // SB

                @pl.when(s + 1 < NSB)
                def _():
                    fire_sb(s + 1)

            @pl.when((g % SB == SB - 1) & (g + 1 < G))
            def _():
                drain_sb()

            @pl.when(g + 1 < G)
            def _():
                stage_a(g + 1, bn)

            rows_wait(b, g)

            @pl.when(g >= 2)
            def _():
                out_wait(b)

            stage_c(g, b)
        return carry

    lax.fori_loop(0, G // 2, main, 0)
    out_wait(0)
    out_wait(1)


@functools.partial(jax.jit, static_argnums=(2, 3, 4))
def _interp(data, lut, npt, c, side, xs, ys, zs):
    PW = npt // NW
    G = PW // L
    KL = K * L
    mesh = plsc.VectorSubcoreMesh(core_axis_name="c", subcore_axis_name="s")
    kern = pl.kernel(
        functools.partial(_body, side, npt, c),
        out_type=jax.ShapeDtypeStruct((npt, c), jnp.float32),
        mesh=mesh,
        scratch_types=[
            pltpu.VMEM((PW,), jnp.float32),
            pltpu.VMEM((PW,), jnp.float32),
            pltpu.VMEM((PW,), jnp.float32),
            pltpu.VMEM((G, KL), jnp.int32),
            pltpu.VMEM((G, KL), jnp.int32),
            pltpu.VMEM((G, KL), jnp.float32),
            pltpu.VMEM((KL,), jnp.int32),
            pltpu.VMEM((KL,), jnp.int32),
            pltpu.VMEM((KL, c), jnp.float32),
            pltpu.VMEM((KL, c), jnp.float32),
            pltpu.VMEM((L, c), jnp.float32),
            pltpu.VMEM((L, c), jnp.float32),
            pltpu.SemaphoreType.DMA,
            pltpu.SemaphoreType.DMA,
            pltpu.SemaphoreType.DMA,
        ],
        compiler_params=pltpu.CompilerParams(
            needs_layout_passes=False, use_tc_tiling_on_sc=False),
    )
    return kern(data, lut, xs, ys, zs)


def kernel(data, octree_lookup, depth, pts):
    del depth
    npt = pts.shape[0]
    c = data.shape[1]
    nvox = octree_lookup.shape[0]
    side = round(nvox ** (1.0 / 3.0))
    assert side ** 3 == nvox and npt % (NW * L) == 0 and c % L == 0
    xs = pts[:, 0]
    ys = pts[:, 1]
    zs = pts[:, 2]
    return _interp(data, octree_lookup, npt, c, side, xs, ys, zs)


# speedup vs baseline: 2.7629x; 1.0006x over previous
import functools

import jax
import jax.numpy as jnp
from jax import lax
from jax.experimental import pallas as pl
from jax.experimental.pallas import tpu as pltpu
from jax.experimental.pallas import tpu_sc as plsc

L = 16
NC = 2
NS = 16
NW = NC * NS
K = 8

_CORNERS = [(dx, dy, dz) for dx in (0, 1) for dy in (0, 1) for dz in (0, 1)]


def _body(side, npt, c, data_hbm, lut_hbm, xs_hbm, ys_hbm, zs_hbm, out_hbm,
          xs_v, ys_v, zs_v, flat_v, node_v, w_v, ridx0_v, ridx1_v,
          rows0_v, rows1_v, out0_v, out1_v, lsem, rsem, osem):
    PW = npt // NW
    G = PW // L
    KL = K * L
    CG = c // L
    scale = side * 0.5

    ridx_b = (ridx0_v, ridx1_v)
    rows_b = (rows0_v, rows1_v)
    out_b = (out0_v, out1_v)

    wid = lax.axis_index("s") * NC + lax.axis_index("c")
    base = wid * PW

    pltpu.sync_copy(xs_hbm.at[pl.ds(base, PW)], xs_v)
    pltpu.sync_copy(ys_hbm.at[pl.ds(base, PW)], ys_v)
    pltpu.sync_copy(zs_hbm.at[pl.ds(base, PW)], zs_v)

    zv = jnp.zeros((L,), jnp.float32)

    def zinit(r, carry):
        for cg in range(CG):
            rows0_v[r, pl.ds(cg * L, L)] = zv
            rows1_v[r, pl.ds(cg * L, L)] = zv
        return carry

    lax.fori_loop(0, KL, zinit, 0)

    def lut_wait():
        pltpu.make_async_copy(lut_hbm.at[flat_v.at[0]], node_v.at[0], lsem).wait()

    def fsplit(v):
        vf = (v + 1.0) * scale - 0.5
        vi = (vf + 1.0).astype(jnp.int32) - 1
        vi = jnp.where(vi.astype(jnp.float32) > vf, vi - 1, vi)
        fr = vf - vi.astype(jnp.float32)
        return vi, fr

    def phase1(g):
        x = xs_v[pl.ds(g * L, L)]
        y = ys_v[pl.ds(g * L, L)]
        z = zs_v[pl.ds(g * L, L)]
        xi, fx = fsplit(x)
        yi, fy = fsplit(y)
        zi, fz = fsplit(z)
        for k, (dx, dy, dz) in enumerate(_CORNERS):
            cx = xi + dx
            cy = yi + dy
            cz = zi + dz
            inb = ((cx >= 0) & (cx < side) & (cy >= 0) & (cy < side)
                   & (cz >= 0) & (cz < side))
            ccx = jnp.clip(cx, 0, side - 1)
            ccy = jnp.clip(cy, 0, side - 1)
            ccz = jnp.clip(cz, 0, side - 1)
            flat = (ccx * side + ccy) * side + ccz
            w = jnp.abs(((1 - dx) - fx) * ((1 - dy) - fy) * ((1 - dz) - fz))
            w = jnp.where(inb, w, 0.0)
            flat_v[g, pl.ds(k * L, L)] = flat
            w_v[g, pl.ds(k * L, L)] = w
        pltpu.async_copy(lut_hbm.at[flat_v.at[g]], node_v.at[g], lsem)

    def stage_a(h, bn):
        for k in range(K):
            nd = node_v[h, pl.ds(k * L, L)]
            valid = nd > -1
            dummy = jnp.bitwise_and(flat_v[h, pl.ds(k * L, L)],
                                    jnp.int32(npt - 1))
            ridx_b[bn][pl.ds(k * L, L)] = jnp.where(valid, nd, dummy)
            wv = w_v[h, pl.ds(k * L, L)]
            w_v[h, pl.ds(k * L, L)] = jnp.where(valid, wv, 0.0)
        pltpu.async_copy(data_hbm.at[ridx_b[bn]], rows_b[bn], rsem)

    def rows_wait(b, g):
        del g
        pltpu.make_async_copy(data_hbm.at[ridx_b[b]], rows_b[b], rsem).wait()

    def stage_c(g, b):
        rows = rows_b[b]
        outv = out_b[b]

        def pbody(p, carry):
            ws = [plsc.load_gather(
                      w_v, [jnp.full((L,), g, jnp.int32),
                            jnp.full((L,), k * L + p, jnp.int32)])
                  for k in range(K)]
            nrm = ws[0]
            for k in range(1, K):
                nrm = nrm + ws[k]
            inv = 1.0 / (nrm + 1e-12)
            for cg in range(CG):
                acc = ws[0] * rows[p, pl.ds(cg * L, L)]
                for k in range(1, K):
                    acc = acc + ws[k] * rows[k * L + p, pl.ds(cg * L, L)]
                outv[p, pl.ds(cg * L, L)] = acc * inv
            return carry

        lax.fori_loop(0, L, pbody, 0)
        pltpu.async_copy(outv, out_hbm.at[pl.ds(base + g * L, L)], osem)

    def out_wait(b):
        pltpu.make_async_copy(out_b[b], out_hbm.at[pl.ds(base, L)], osem).wait()

    SB = 16
    NSB = G // SB

    def fire_sb(s):
        def fb(g, carry):
            phase1(g)
            return carry
        lax.fori_loop(s * SB, (s + 1) * SB, fb, 0)

    def drain_sb():
        def db(_, carry):
            lut_wait()
            return carry
        lax.fori_loop(0, SB, db, 0)

    fire_sb(0)
    drain_sb()
    stage_a(0, 0)

    def main(i, carry):
        for off in range(2):
            g = i * 2 + off
            b = off
            bn = 1 - off

            @pl.when(g % SB == 0)
            def _():
                s = g // SB

                @pl.when(s + 1 < NSB)
                def _():
                    fire_sb(s + 1)

            @pl.when((g % SB == SB - 1) & (g + 1 < G))
            def _():
                drain_sb()

            @pl.when(g + 1 < G)
            def _():
                stage_a(g + 1, bn)

            rows_wait(b, g)

            @pl.when(g >= 2)
            def _():
                out_wait(b)

            stage_c(g, b)
        return carry

    lax.fori_loop(0, G // 2, main, 0)
    out_wait(0)
    out_wait(1)


@functools.partial(jax.jit, static_argnums=(2, 3, 4))
def _interp(data, lut, npt, c, side, xs, ys, zs):
    PW = npt // NW
    G = PW // L
    KL = K * L
    mesh = plsc.VectorSubcoreMesh(core_axis_name="c", subcore_axis_name="s")
    kern = pl.kernel(
        functools.partial(_body, side, npt, c),
        out_type=jax.ShapeDtypeStruct((npt, c), jnp.float32),
        mesh=mesh,
        scratch_types=[
            pltpu.VMEM((PW,), jnp.float32),
            pltpu.VMEM((PW,), jnp.float32),
            pltpu.VMEM((PW,), jnp.float32),
            pltpu.VMEM((G, KL), jnp.int32),
            pltpu.VMEM((G, KL), jnp.int32),
            pltpu.VMEM((G, KL), jnp.float32),
            pltpu.VMEM((KL,), jnp.int32),
            pltpu.VMEM((KL,), jnp.int32),
            pltpu.VMEM((KL, c), jnp.float32),
            pltpu.VMEM((KL, c), jnp.float32),
            pltpu.VMEM((L, c), jnp.float32),
            pltpu.VMEM((L, c), jnp.float32),
            pltpu.SemaphoreType.DMA,
            pltpu.SemaphoreType.DMA,
            pltpu.SemaphoreType.DMA,
        ],
        compiler_params=pltpu.CompilerParams(
            needs_layout_passes=False, use_tc_tiling_on_sc=False),
    )
    return kern(data, lut, xs, ys, zs)


def kernel(data, octree_lookup, depth, pts):
    del depth
    npt = pts.shape[0]
    c = data.shape[1]
    nvox = octree_lookup.shape[0]
    side = round(nvox ** (1.0 / 3.0))
    assert side ** 3 == nvox and npt % (NW * L) == 0 and c % L == 0
    xs = pts[:, 0]
    ys = pts[:, 1]
    zs = pts[:, 2]
    return _interp(data, octree_lookup, npt, c, side, xs, ys, zs)
